# baseline (speedup 1.0000x reference)
import functools

import jax
import jax.numpy as jnp
from jax import lax
from jax.experimental import pallas as pl
from jax.experimental.pallas import tpu as pltpu
from jax.experimental.pallas import tpu_sc as plsc

N = 10000
E = 320000
D = 128
ALPHA = 0.5

NPAD = 10240
MC = 512
TCHM = 40
NCHM = 16 * TCHM
EPAD = NCHM * MC
RPT = NPAD // 16

_mesh = plsc.VectorSubcoreMesh(core_axis_name="c", subcore_axis_name="s")


@functools.partial(
    pl.kernel,
    out_type=jax.ShapeDtypeStruct((2, NPAD, 64), jnp.float32),
    mesh=_mesh,
    scratch_types=[
        pltpu.VMEM((TCHM, MC), jnp.int32),
        pltpu.VMEM((MC, 64), jnp.float32),
        pltpu.VMEM_SHARED((NPAD, 64), jnp.float32),
        pltpu.SemaphoreType.DMA,
    ],
    compiler_params=pltpu.CompilerParams(use_tc_tiling_on_sc=False),
)
def _hist(sidx_hbm, dout_hbm, idx_v, fill_v, dacc, sem):
    c = lax.axis_index("c")
    s = lax.axis_index("s")

    def zrow(i, carry):
        for k in range(4):
            fill_v[i, pl.ds(k * 16, 16)] = jnp.zeros((16,), jnp.float32)
        return carry

    lax.fori_loop(0, MC, zrow, 0)
    pltpu.sync_copy(fill_v, dacc.at[pl.ds(s * RPT, MC)])
    pltpu.sync_copy(fill_v.at[pl.ds(0, RPT - MC)],
                    dacc.at[pl.ds(s * RPT + MC, RPT - MC)])

    def orow(i, carry):
        for k in range(4):
            fill_v[i, pl.ds(k * 16, 16)] = jnp.ones((16,), jnp.float32)
        return carry

    lax.fori_loop(0, MC, orow, 0)

    pltpu.sync_copy(sidx_hbm.at[c, pl.ds(s * TCHM, TCHM)], idx_v)
    plsc.subcore_barrier()

    K = 4

    def body(i, carry):
        pltpu.async_copy(fill_v, dacc.at[idx_v.at[i]], sem, add=True)

        @pl.when(i >= K - 1)
        def _():
            pltpu.make_async_copy(fill_v, dacc.at[idx_v.at[0]], sem).wait()

        return carry

    lax.fori_loop(0, TCHM, body, 0)
    for _ in range(K - 1):
        pltpu.make_async_copy(fill_v, dacc.at[idx_v.at[0]], sem).wait()
    plsc.subcore_barrier()
    pltpu.sync_copy(dacc.at[pl.ds(s * RPT, RPT)],
                    dout_hbm.at[c, pl.ds(s * RPT, RPT)])


_BLK = 1024


def _scale_body(dp_ref, x_ref, inv_ref, t_ref):
    dp = dp_ref[...]
    x = x_ref[...]
    deg = jnp.sum(dp, axis=-1) * (1.0 / 64.0)
    inv = jnp.where(deg > 0, lax.rsqrt(jnp.maximum(deg, 1.0)), 0.0)
    inv_ref[...] = inv
    t_ref[0] = inv[1][:, None] * x
    t_ref[1] = inv[0][:, None] * x


_scale = pl.pallas_call(
    _scale_body,
    grid=(NPAD // _BLK,),
    in_specs=[
        pl.BlockSpec((2, _BLK, 64), lambda i: (0, i, 0)),
        pl.BlockSpec((_BLK, D), lambda i: (i, 0)),
    ],
    out_specs=[
        pl.BlockSpec((2, _BLK), lambda i: (0, i)),
        pl.BlockSpec((2, _BLK, D), lambda i: (0, i, 0)),
    ],
    out_shape=[
        jax.ShapeDtypeStruct((2, NPAD), jnp.float32),
        jax.ShapeDtypeStruct((2, NPAD, D), jnp.float32),
    ],
)


DH = D // 2


@functools.partial(
    pl.kernel,
    out_type=jax.ShapeDtypeStruct((2, 2, NPAD, DH), jnp.float32),
    mesh=_mesh,
    scratch_types=[
        pltpu.VMEM((TCHM, MC), jnp.int32),
        pltpu.VMEM((TCHM, MC), jnp.int32),
        pltpu.VMEM((MC, DH), jnp.float32),
        pltpu.VMEM_SHARED((NPAD, DH), jnp.float32),
        pltpu.SemaphoreType.DMA,
    ],
    compiler_params=pltpu.CompilerParams(use_tc_tiling_on_sc=False),
)
def _scat(tab_hbm, gidx_hbm, sidx_hbm, out_hbm,
          gidx_v, sidx_v, buf, acc, sem):
    c = lax.axis_index("c")
    s = lax.axis_index("s")

    pltpu.sync_copy(gidx_hbm.at[c, pl.ds(s * TCHM, TCHM)], gidx_v)
    pltpu.sync_copy(sidx_hbm.at[c, pl.ds(s * TCHM, TCHM)], sidx_v)

    for h in range(2):
        def zrow(i, carry):
            for k in range(DH // 16):
                buf[i, pl.ds(k * 16, 16)] = jnp.zeros((16,), jnp.float32)
            return carry

        lax.fori_loop(0, MC, zrow, 0)
        pltpu.sync_copy(buf, acc.at[pl.ds(s * RPT, MC)])
        pltpu.sync_copy(buf.at[pl.ds(0, RPT - MC)],
                        acc.at[pl.ds(s * RPT + MC, RPT - MC)])
        if h == 1:
            def bump(i, carry):
                for k in range(MC // 16):
                    sl = pl.ds(k * 16, 16)
                    gidx_v[i, sl] = gidx_v[i, sl] + 1
                return carry

            lax.fori_loop(0, TCHM, bump, 0)
        plsc.subcore_barrier()

        def body(i, carry):
            pltpu.async_copy(tab_hbm.at[gidx_v.at[i]], buf, sem).wait()
            pltpu.sync_copy(buf, acc.at[sidx_v.at[i]], add=True)
            return carry

        lax.fori_loop(0, TCHM, body, 0)
        plsc.subcore_barrier()
        pltpu.sync_copy(acc.at[pl.ds(s * RPT, RPT)],
                        out_hbm.at[c, h, pl.ds(s * RPT, RPT)])


def _out_body(s_ref, inv_ref, ws_ref, wd_ref, bs_ref, bd_ref, o_ref):
    s0 = lax.concatenate([s_ref[0, 0], s_ref[0, 1]], 1)
    s1 = lax.concatenate([s_ref[1, 0], s_ref[1, 1]], 1)
    inv = inv_ref[...]
    y0 = lax.dot_general(s0, ws_ref[...], (((1,), (1,)), ((), ())),
                         preferred_element_type=jnp.float32)
    y1 = lax.dot_general(s1, wd_ref[...], (((1,), (1,)), ((), ())),
                         preferred_element_type=jnp.float32)
    o_ref[...] = (ALPHA * inv[0][:, None] * y0
                  + (1.0 - ALPHA) * inv[1][:, None] * y1
                  + ALPHA * bs_ref[...] + (1.0 - ALPHA) * bd_ref[...])


_outk = pl.pallas_call(
    _out_body,
    grid=(NPAD // _BLK,),
    in_specs=[
        pl.BlockSpec((2, 2, _BLK, DH), lambda i: (0, 0, i, 0)),
        pl.BlockSpec((2, _BLK), lambda i: (0, i)),
        pl.BlockSpec((D, D), lambda i: (0, 0)),
        pl.BlockSpec((D, D), lambda i: (0, 0)),
        pl.BlockSpec((1, D), lambda i: (0, 0)),
        pl.BlockSpec((1, D), lambda i: (0, 0)),
    ],
    out_specs=pl.BlockSpec((_BLK, D), lambda i: (i, 0)),
    out_shape=jax.ShapeDtypeStruct((NPAD, D), jnp.float32),
)


def kernel(x, edge_index, W_src, b_src, W_dst, b_dst):
    x_pad = jnp.pad(x, ((0, NPAD - N), (0, 0)))
    npad_e = EPAD - E
    sidx3 = jnp.pad(edge_index, ((0, 0), (0, npad_e)),
                    constant_values=NPAD - 1).reshape(2, NCHM, MC)
    gsrc = jnp.pad(edge_index[::-1], ((0, 0), (0, npad_e)))
    gidx3 = (2 * (gsrc + jnp.array([[0], [NPAD]], jnp.int32))
             ).reshape(2, NCHM, MC)
    dpart = _hist(sidx3)
    inv, t = _scale(dpart, x_pad)
    tab = t.reshape(4 * NPAD, DH)
    svals = _scat(tab, gidx3, sidx3)
    out = _outk(svals, inv, W_src, W_dst,
                b_src.reshape(1, D), b_dst.reshape(1, D))
    return out[:N]


# speedup vs baseline: 1.0560x; 1.0560x over previous
import functools

import jax
import jax.numpy as jnp
from jax import lax
from jax.experimental import pallas as pl
from jax.experimental.pallas import tpu as pltpu
from jax.experimental.pallas import tpu_sc as plsc

N = 10000
E = 320000
D = 128
ALPHA = 0.5

NPAD = 10240
MC = 512
TCHM = 40
NCHM = 16 * TCHM
EPAD = NCHM * MC
RPT = NPAD // 16

_mesh = plsc.VectorSubcoreMesh(core_axis_name="c", subcore_axis_name="s")


@functools.partial(
    pl.kernel,
    out_type=jax.ShapeDtypeStruct((2, NPAD, 64), jnp.float32),
    mesh=_mesh,
    scratch_types=[
        pltpu.VMEM((TCHM, MC), jnp.int32),
        pltpu.VMEM((MC, 64), jnp.float32),
        pltpu.VMEM_SHARED((NPAD, 64), jnp.float32),
        pltpu.SemaphoreType.DMA,
    ],
    compiler_params=pltpu.CompilerParams(use_tc_tiling_on_sc=False),
)
def _hist(sidx_hbm, dout_hbm, idx_v, fill_v, dacc, sem):
    c = lax.axis_index("c")
    s = lax.axis_index("s")

    def zrow(i, carry):
        for k in range(4):
            fill_v[i, pl.ds(k * 16, 16)] = jnp.zeros((16,), jnp.float32)
        return carry

    lax.fori_loop(0, MC, zrow, 0)
    pltpu.sync_copy(fill_v, dacc.at[pl.ds(s * RPT, MC)])
    pltpu.sync_copy(fill_v.at[pl.ds(0, RPT - MC)],
                    dacc.at[pl.ds(s * RPT + MC, RPT - MC)])

    def orow(i, carry):
        for k in range(4):
            fill_v[i, pl.ds(k * 16, 16)] = jnp.ones((16,), jnp.float32)
        return carry

    lax.fori_loop(0, MC, orow, 0)

    pltpu.sync_copy(sidx_hbm.at[c, pl.ds(s * TCHM, TCHM)], idx_v)
    plsc.subcore_barrier()

    K = 4

    def body(i, carry):
        pltpu.async_copy(fill_v, dacc.at[idx_v.at[i]], sem, add=True)

        @pl.when(i >= K - 1)
        def _():
            pltpu.make_async_copy(fill_v, dacc.at[idx_v.at[0]], sem).wait()

        return carry

    lax.fori_loop(0, TCHM, body, 0)
    for _ in range(K - 1):
        pltpu.make_async_copy(fill_v, dacc.at[idx_v.at[0]], sem).wait()
    plsc.subcore_barrier()
    pltpu.sync_copy(dacc.at[pl.ds(s * RPT, RPT)],
                    dout_hbm.at[c, pl.ds(s * RPT, RPT)])


_BLK = 1024


def _scale_body(dp_ref, x_ref, inv_ref, t_ref):
    dp = dp_ref[...]
    x = x_ref[...]
    deg = jnp.sum(dp, axis=-1) * (1.0 / 64.0)
    inv = jnp.where(deg > 0, lax.rsqrt(jnp.maximum(deg, 1.0)), 0.0)
    inv_ref[...] = inv
    t_ref[0] = inv[1][:, None] * x
    t_ref[1] = inv[0][:, None] * x


_scale = pl.pallas_call(
    _scale_body,
    grid=(NPAD // _BLK,),
    in_specs=[
        pl.BlockSpec((2, _BLK, 64), lambda i: (0, i, 0)),
        pl.BlockSpec((_BLK, D), lambda i: (i, 0)),
    ],
    out_specs=[
        pl.BlockSpec((2, _BLK), lambda i: (0, i)),
        pl.BlockSpec((2, _BLK, D), lambda i: (0, i, 0)),
    ],
    out_shape=[
        jax.ShapeDtypeStruct((2, NPAD), jnp.float32),
        jax.ShapeDtypeStruct((2, NPAD, D), jnp.float32),
    ],
)


DH = D // 2
MCS = 256
TCHS = EPAD // (16 * MCS)
NCHS = 16 * TCHS


@functools.partial(
    pl.kernel,
    out_type=jax.ShapeDtypeStruct((2, 2, NPAD, DH), jnp.float32),
    mesh=_mesh,
    scratch_types=[
        pltpu.VMEM((TCHS, MCS), jnp.int32),
        pltpu.VMEM((TCHS, MCS), jnp.int32),
        [pltpu.VMEM((MCS, DH), jnp.float32)] * 2,
        pltpu.VMEM_SHARED((NPAD, DH), jnp.float32),
        [pltpu.SemaphoreType.DMA] * 2,
        [pltpu.SemaphoreType.DMA] * 2,
    ],
    compiler_params=pltpu.CompilerParams(use_tc_tiling_on_sc=False),
)
def _scat(tab_hbm, gidx_hbm, sidx_hbm, out_hbm,
          gidx_v, sidx_v, bufs, acc, gsems, ssems):
    c = lax.axis_index("c")
    s = lax.axis_index("s")

    pltpu.sync_copy(gidx_hbm.at[c, pl.ds(s * TCHS, TCHS)], gidx_v)
    pltpu.sync_copy(sidx_hbm.at[c, pl.ds(s * TCHS, TCHS)], sidx_v)

    NB = 2

    for h in range(2):
        def zrow(i, carry):
            for k in range(DH // 16):
                bufs[0][i, pl.ds(k * 16, 16)] = jnp.zeros((16,), jnp.float32)
            return carry

        lax.fori_loop(0, MCS, zrow, 0)
        for j in range(RPT // MCS + (1 if RPT % MCS else 0)):
            w = min(MCS, RPT - j * MCS)
            pltpu.sync_copy(bufs[0].at[pl.ds(0, w)],
                            acc.at[pl.ds(s * RPT + j * MCS, w)])
        if h == 1:
            def bump(i, carry):
                for k in range(MCS // 16):
                    sl = pl.ds(k * 16, 16)
                    gidx_v[i, sl] = gidx_v[i, sl] + 1
                return carry

            lax.fori_loop(0, TCHS, bump, 0)
        plsc.subcore_barrier()

        def grp(g, carry):
            i0 = NB * g
            for b in range(NB):
                @pl.when(g > 0)
                def _(b=b):
                    pltpu.make_async_copy(
                        bufs[b], acc.at[sidx_v.at[0]], ssems[b]).wait()

                pltpu.async_copy(tab_hbm.at[gidx_v.at[i0 + b]],
                                 bufs[b], gsems[b])
            for b in range(NB):
                pltpu.make_async_copy(tab_hbm.at[gidx_v.at[i0 + b]],
                                      bufs[b], gsems[b]).wait()
                pltpu.async_copy(bufs[b], acc.at[sidx_v.at[i0 + b]],
                                 ssems[b], add=True)
            return carry

        lax.fori_loop(0, TCHS // NB, grp, 0)
        for b in range(NB):
            pltpu.make_async_copy(bufs[b], acc.at[sidx_v.at[0]],
                                  ssems[b]).wait()
        plsc.subcore_barrier()
        pltpu.sync_copy(acc.at[pl.ds(s * RPT, RPT)],
                        out_hbm.at[c, h, pl.ds(s * RPT, RPT)])


def _out_body(s_ref, inv_ref, ws_ref, wd_ref, bs_ref, bd_ref, o_ref):
    s0 = lax.concatenate([s_ref[0, 0], s_ref[0, 1]], 1)
    s1 = lax.concatenate([s_ref[1, 0], s_ref[1, 1]], 1)
    inv = inv_ref[...]
    y0 = lax.dot_general(s0, ws_ref[...], (((1,), (1,)), ((), ())),
                         preferred_element_type=jnp.float32)
    y1 = lax.dot_general(s1, wd_ref[...], (((1,), (1,)), ((), ())),
                         preferred_element_type=jnp.float32)
    o_ref[...] = (ALPHA * inv[0][:, None] * y0
                  + (1.0 - ALPHA) * inv[1][:, None] * y1
                  + ALPHA * bs_ref[...] + (1.0 - ALPHA) * bd_ref[...])


_outk = pl.pallas_call(
    _out_body,
    grid=(NPAD // _BLK,),
    in_specs=[
        pl.BlockSpec((2, 2, _BLK, DH), lambda i: (0, 0, i, 0)),
        pl.BlockSpec((2, _BLK), lambda i: (0, i)),
        pl.BlockSpec((D, D), lambda i: (0, 0)),
        pl.BlockSpec((D, D), lambda i: (0, 0)),
        pl.BlockSpec((1, D), lambda i: (0, 0)),
        pl.BlockSpec((1, D), lambda i: (0, 0)),
    ],
    out_specs=pl.BlockSpec((_BLK, D), lambda i: (i, 0)),
    out_shape=jax.ShapeDtypeStruct((NPAD, D), jnp.float32),
)


def kernel(x, edge_index, W_src, b_src, W_dst, b_dst):
    x_pad = jnp.pad(x, ((0, NPAD - N), (0, 0)))
    npad_e = EPAD - E
    sidx_flat = jnp.pad(edge_index, ((0, 0), (0, npad_e)),
                        constant_values=NPAD - 1)
    gsrc = jnp.pad(edge_index[::-1], ((0, 0), (0, npad_e)))
    gidx_flat = 2 * (gsrc + jnp.array([[0], [NPAD]], jnp.int32))
    sidx3 = sidx_flat.reshape(2, NCHS, MCS)
    gidx3 = gidx_flat.reshape(2, NCHS, MCS)
    dpart = _hist(sidx_flat.reshape(2, NCHM, MC))
    inv, t = _scale(dpart, x_pad)
    tab = t.reshape(4 * NPAD, DH)
    svals = _scat(tab, gidx3, sidx3)
    out = _outk(svals, inv, W_src, W_dst,
                b_src.reshape(1, D), b_dst.reshape(1, D))
    return out[:N]


# speedup vs baseline: 1.1357x; 1.0755x over previous
import functools

import jax
import jax.numpy as jnp
from jax import lax
from jax.experimental import pallas as pl
from jax.experimental.pallas import tpu as pltpu
from jax.experimental.pallas import tpu_sc as plsc

N = 10000
E = 320000
D = 128
ALPHA = 0.5

NPAD = 10240
MC = 512
TCHM = 40
NCHM = 16 * TCHM
EPAD = NCHM * MC
RPT = NPAD // 16

_mesh = plsc.VectorSubcoreMesh(core_axis_name="c", subcore_axis_name="s")


@functools.partial(
    pl.kernel,
    out_type=jax.ShapeDtypeStruct((2, NPAD, 16), jnp.float32),
    mesh=_mesh,
    scratch_types=[
        pltpu.VMEM((TCHM, MC), jnp.int32),
        pltpu.VMEM((MC, 16), jnp.float32),
        pltpu.VMEM_SHARED((NPAD, 16), jnp.float32),
        pltpu.SemaphoreType.DMA,
    ],
    compiler_params=pltpu.CompilerParams(use_tc_tiling_on_sc=False),
)
def _hist(sidx_hbm, dout_hbm, idx_v, fill_v, dacc, sem):
    c = lax.axis_index("c")
    s = lax.axis_index("s")

    def zrow(i, carry):
        fill_v[i, :] = jnp.zeros((16,), jnp.float32)
        return carry

    lax.fori_loop(0, MC, zrow, 0)
    pltpu.sync_copy(fill_v, dacc.at[pl.ds(s * RPT, MC)])
    pltpu.sync_copy(fill_v.at[pl.ds(0, RPT - MC)],
                    dacc.at[pl.ds(s * RPT + MC, RPT - MC)])

    def orow(i, carry):
        fill_v[i, :] = jnp.ones((16,), jnp.float32)
        return carry

    lax.fori_loop(0, MC, orow, 0)

    pltpu.sync_copy(sidx_hbm.at[c, pl.ds(s * TCHM, TCHM)], idx_v)
    plsc.subcore_barrier()

    K = 4

    def body(i, carry):
        pltpu.async_copy(fill_v, dacc.at[idx_v.at[i]], sem, add=True)

        @pl.when(i >= K - 1)
        def _():
            pltpu.make_async_copy(fill_v, dacc.at[idx_v.at[0]], sem).wait()

        return carry

    lax.fori_loop(0, TCHM, body, 0)
    for _ in range(K - 1):
        pltpu.make_async_copy(fill_v, dacc.at[idx_v.at[0]], sem).wait()
    plsc.subcore_barrier()
    pltpu.sync_copy(dacc.at[pl.ds(s * RPT, RPT)],
                    dout_hbm.at[c, pl.ds(s * RPT, RPT)])


_BLK = 1024


def _scale_body(dp_ref, x_ref, inv_ref, t_ref):
    dp = dp_ref[...]
    x = x_ref[...]
    deg = jnp.sum(dp, axis=-1) * (1.0 / 16.0)
    inv = jnp.where(deg > 0, lax.rsqrt(jnp.maximum(deg, 1.0)), 0.0)
    inv_ref[...] = inv
    t_ref[0] = inv[1][:, None] * x
    t_ref[1] = inv[0][:, None] * x


_scale = pl.pallas_call(
    _scale_body,
    grid=(NPAD // _BLK,),
    in_specs=[
        pl.BlockSpec((2, _BLK, 16), lambda i: (0, i, 0)),
        pl.BlockSpec((_BLK, D), lambda i: (i, 0)),
    ],
    out_specs=[
        pl.BlockSpec((2, _BLK), lambda i: (0, i)),
        pl.BlockSpec((2, _BLK, D), lambda i: (0, i, 0)),
    ],
    out_shape=[
        jax.ShapeDtypeStruct((2, NPAD), jnp.float32),
        jax.ShapeDtypeStruct((2, NPAD, D), jnp.float32),
    ],
)


DH = D // 2
MCS = 128
TCHS = EPAD // (16 * MCS)
NCHS = 16 * TCHS


@functools.partial(
    pl.kernel,
    out_type=jax.ShapeDtypeStruct((2, 2, NPAD, DH), jnp.float32),
    mesh=_mesh,
    scratch_types=[
        pltpu.VMEM((TCHS, MCS), jnp.int32),
        pltpu.VMEM((TCHS, MCS), jnp.int32),
        [pltpu.VMEM((MCS, DH), jnp.float32)] * 4,
        pltpu.VMEM_SHARED((NPAD, DH), jnp.float32),
        [pltpu.SemaphoreType.DMA] * 4,
        [pltpu.SemaphoreType.DMA] * 4,
    ],
    compiler_params=pltpu.CompilerParams(use_tc_tiling_on_sc=False),
)
def _scat(tab_hbm, gidx_hbm, sidx_hbm, out_hbm,
          gidx_v, sidx_v, bufs, acc, gsems, ssems):
    c = lax.axis_index("c")
    s = lax.axis_index("s")

    pltpu.sync_copy(gidx_hbm.at[c, pl.ds(s * TCHS, TCHS)], gidx_v)
    pltpu.sync_copy(sidx_hbm.at[c, pl.ds(s * TCHS, TCHS)], sidx_v)

    NB = 4

    for h in range(2):
        def zrow(i, carry):
            for k in range(DH // 16):
                bufs[0][i, pl.ds(k * 16, 16)] = jnp.zeros((16,), jnp.float32)
            return carry

        lax.fori_loop(0, MCS, zrow, 0)
        for j in range(RPT // MCS + (1 if RPT % MCS else 0)):
            w = min(MCS, RPT - j * MCS)
            pltpu.sync_copy(bufs[0].at[pl.ds(0, w)],
                            acc.at[pl.ds(s * RPT + j * MCS, w)])
        if h == 1:
            def bump(i, carry):
                for k in range(MCS // 16):
                    sl = pl.ds(k * 16, 16)
                    gidx_v[i, sl] = gidx_v[i, sl] + 1
                return carry

            lax.fori_loop(0, TCHS, bump, 0)
        plsc.subcore_barrier()

        def grp(g, carry):
            i0 = NB * g
            for b in range(NB):
                @pl.when(g > 0)
                def _(b=b):
                    pltpu.make_async_copy(
                        bufs[b], acc.at[sidx_v.at[0]], ssems[b]).wait()

                pltpu.async_copy(tab_hbm.at[gidx_v.at[i0 + b]],
                                 bufs[b], gsems[b])
            for b in range(NB):
                pltpu.make_async_copy(tab_hbm.at[gidx_v.at[i0 + b]],
                                      bufs[b], gsems[b]).wait()
                pltpu.async_copy(bufs[b], acc.at[sidx_v.at[i0 + b]],
                                 ssems[b], add=True)
            return carry

        lax.fori_loop(0, TCHS // NB, grp, 0)
        for b in range(NB):
            pltpu.make_async_copy(bufs[b], acc.at[sidx_v.at[0]],
                                  ssems[b]).wait()
        plsc.subcore_barrier()
        pltpu.sync_copy(acc.at[pl.ds(s * RPT, RPT)],
                        out_hbm.at[c, h, pl.ds(s * RPT, RPT)])


def _out_body(s_ref, inv_ref, ws_ref, wd_ref, bs_ref, bd_ref, o_ref):
    s0 = lax.concatenate([s_ref[0, 0], s_ref[0, 1]], 1)
    s1 = lax.concatenate([s_ref[1, 0], s_ref[1, 1]], 1)
    inv = inv_ref[...]
    y0 = lax.dot_general(s0, ws_ref[...], (((1,), (1,)), ((), ())),
                         preferred_element_type=jnp.float32)
    y1 = lax.dot_general(s1, wd_ref[...], (((1,), (1,)), ((), ())),
                         preferred_element_type=jnp.float32)
    o_ref[...] = (ALPHA * inv[0][:, None] * y0
                  + (1.0 - ALPHA) * inv[1][:, None] * y1
                  + ALPHA * bs_ref[...] + (1.0 - ALPHA) * bd_ref[...])


_outk = pl.pallas_call(
    _out_body,
    grid=(NPAD // _BLK,),
    in_specs=[
        pl.BlockSpec((2, 2, _BLK, DH), lambda i: (0, 0, i, 0)),
        pl.BlockSpec((2, _BLK), lambda i: (0, i)),
        pl.BlockSpec((D, D), lambda i: (0, 0)),
        pl.BlockSpec((D, D), lambda i: (0, 0)),
        pl.BlockSpec((1, D), lambda i: (0, 0)),
        pl.BlockSpec((1, D), lambda i: (0, 0)),
    ],
    out_specs=pl.BlockSpec((_BLK, D), lambda i: (i, 0)),
    out_shape=jax.ShapeDtypeStruct((NPAD, D), jnp.float32),
)


def kernel(x, edge_index, W_src, b_src, W_dst, b_dst):
    x_pad = jnp.pad(x, ((0, NPAD - N), (0, 0)))
    npad_e = EPAD - E
    sidx_flat = jnp.pad(edge_index, ((0, 0), (0, npad_e)),
                        constant_values=NPAD - 1)
    gsrc = jnp.pad(edge_index[::-1], ((0, 0), (0, npad_e)))
    gidx_flat = 2 * (gsrc + jnp.array([[0], [NPAD]], jnp.int32))
    sidx3 = sidx_flat.reshape(2, NCHS, MCS)
    gidx3 = gidx_flat.reshape(2, NCHS, MCS)
    dpart = _hist(sidx_flat.reshape(2, NCHM, MC))
    inv, t = _scale(dpart, x_pad)
    tab = t.reshape(4 * NPAD, DH)
    svals = _scat(tab, gidx3, sidx3)
    out = _outk(svals, inv, W_src, W_dst,
                b_src.reshape(1, D), b_dst.reshape(1, D))
    return out[:N]


# speedup vs baseline: 1.1449x; 1.0081x over previous
import functools

import jax
import jax.numpy as jnp
from jax import lax
from jax.experimental import pallas as pl
from jax.experimental.pallas import tpu as pltpu
from jax.experimental.pallas import tpu_sc as plsc

N = 10000
E = 320000
D = 128
ALPHA = 0.5

NPAD = 10240
MC = 512
TCHM = 40
NCHM = 16 * TCHM
EPAD = NCHM * MC
RPT = NPAD // 16

_mesh = plsc.VectorSubcoreMesh(core_axis_name="c", subcore_axis_name="s")


@functools.partial(
    pl.kernel,
    out_type=jax.ShapeDtypeStruct((2, NPAD, 16), jnp.float32),
    mesh=_mesh,
    scratch_types=[
        pltpu.VMEM((TCHM, MC), jnp.int32),
        pltpu.VMEM((MC, 16), jnp.float32),
        pltpu.VMEM_SHARED((NPAD, 16), jnp.float32),
        pltpu.SemaphoreType.DMA,
    ],
    compiler_params=pltpu.CompilerParams(use_tc_tiling_on_sc=False),
)
def _hist(sidx_hbm, dout_hbm, idx_v, fill_v, dacc, sem):
    c = lax.axis_index("c")
    s = lax.axis_index("s")

    def zrow(i, carry):
        fill_v[i, :] = jnp.zeros((16,), jnp.float32)
        return carry

    lax.fori_loop(0, MC, zrow, 0)
    pltpu.sync_copy(fill_v, dacc.at[pl.ds(s * RPT, MC)])
    pltpu.sync_copy(fill_v.at[pl.ds(0, RPT - MC)],
                    dacc.at[pl.ds(s * RPT + MC, RPT - MC)])

    def orow(i, carry):
        fill_v[i, :] = jnp.ones((16,), jnp.float32)
        return carry

    lax.fori_loop(0, MC, orow, 0)

    pltpu.sync_copy(sidx_hbm.at[c, pl.ds(s * TCHM, TCHM)], idx_v)
    plsc.subcore_barrier()

    K = 4

    def body(i, carry):
        pltpu.async_copy(fill_v, dacc.at[idx_v.at[i]], sem, add=True)

        @pl.when(i >= K - 1)
        def _():
            pltpu.make_async_copy(fill_v, dacc.at[idx_v.at[0]], sem).wait()

        return carry

    lax.fori_loop(0, TCHM, body, 0)
    for _ in range(K - 1):
        pltpu.make_async_copy(fill_v, dacc.at[idx_v.at[0]], sem).wait()
    plsc.subcore_barrier()
    pltpu.sync_copy(dacc.at[pl.ds(s * RPT, RPT)],
                    dout_hbm.at[c, pl.ds(s * RPT, RPT)])


_BLK = 1024


def _scale_body(dp_ref, x_ref, inv_ref, t_ref):
    dp = dp_ref[...]
    x = x_ref[...]
    deg = jnp.sum(dp, axis=-1) * (1.0 / 16.0)
    inv = jnp.where(deg > 0, lax.rsqrt(jnp.maximum(deg, 1.0)), 0.0)
    inv_ref[...] = inv
    t_ref[0] = inv[1][:, None] * x
    t_ref[1] = inv[0][:, None] * x


_scale = pl.pallas_call(
    _scale_body,
    grid=(NPAD // _BLK,),
    in_specs=[
        pl.BlockSpec((2, _BLK, 16), lambda i: (0, i, 0)),
        pl.BlockSpec((_BLK, D), lambda i: (i, 0)),
    ],
    out_specs=[
        pl.BlockSpec((2, _BLK), lambda i: (0, i)),
        pl.BlockSpec((2, _BLK, D), lambda i: (0, i, 0)),
    ],
    out_shape=[
        jax.ShapeDtypeStruct((2, NPAD), jnp.float32),
        jax.ShapeDtypeStruct((2, NPAD, D), jnp.float32),
    ],
)


DH = D // 2
MCS = 160
TCHS = EPAD // (16 * MCS)
NCHS = 16 * TCHS


@functools.partial(
    pl.kernel,
    out_type=jax.ShapeDtypeStruct((2, 2, NPAD, DH), jnp.float32),
    mesh=_mesh,
    scratch_types=[
        pltpu.VMEM((TCHS, MCS), jnp.int32),
        pltpu.VMEM((TCHS, MCS), jnp.int32),
        [pltpu.VMEM((MCS, DH), jnp.float32)] * 4,
        pltpu.VMEM_SHARED((NPAD, DH), jnp.float32),
        [pltpu.SemaphoreType.DMA] * 4,
        [pltpu.SemaphoreType.DMA] * 4,
    ],
    compiler_params=pltpu.CompilerParams(use_tc_tiling_on_sc=False),
)
def _scat(tab_hbm, gidx_hbm, sidx_hbm, out_hbm,
          gidx_v, sidx_v, bufs, acc, gsems, ssems):
    c = lax.axis_index("c")
    s = lax.axis_index("s")

    pltpu.sync_copy(gidx_hbm.at[c, pl.ds(s * TCHS, TCHS)], gidx_v)
    pltpu.sync_copy(sidx_hbm.at[c, pl.ds(s * TCHS, TCHS)], sidx_v)

    NB = 4

    for h in range(2):
        def zrow(i, carry):
            for k in range(DH // 16):
                bufs[0][i, pl.ds(k * 16, 16)] = jnp.zeros((16,), jnp.float32)
            return carry

        lax.fori_loop(0, MCS, zrow, 0)
        for j in range(RPT // MCS + (1 if RPT % MCS else 0)):
            w = min(MCS, RPT - j * MCS)
            pltpu.sync_copy(bufs[0].at[pl.ds(0, w)],
                            acc.at[pl.ds(s * RPT + j * MCS, w)])
        if h == 1:
            def bump(i, carry):
                for k in range(MCS // 16):
                    sl = pl.ds(k * 16, 16)
                    gidx_v[i, sl] = gidx_v[i, sl] + 1
                return carry

            lax.fori_loop(0, TCHS, bump, 0)
        plsc.subcore_barrier()

        def grp(g, carry):
            i0 = NB * g
            for b in range(NB):
                @pl.when(g > 0)
                def _(b=b):
                    pltpu.make_async_copy(
                        bufs[b], acc.at[sidx_v.at[0]], ssems[b]).wait()

                pltpu.async_copy(tab_hbm.at[gidx_v.at[i0 + b]],
                                 bufs[b], gsems[b])
            for b in range(NB):
                pltpu.make_async_copy(tab_hbm.at[gidx_v.at[i0 + b]],
                                      bufs[b], gsems[b]).wait()
                pltpu.async_copy(bufs[b], acc.at[sidx_v.at[i0 + b]],
                                 ssems[b], add=True)
            return carry

        lax.fori_loop(0, TCHS // NB, grp, 0)
        for b in range(NB):
            pltpu.make_async_copy(bufs[b], acc.at[sidx_v.at[0]],
                                  ssems[b]).wait()
        plsc.subcore_barrier()
        pltpu.sync_copy(acc.at[pl.ds(s * RPT, RPT)],
                        out_hbm.at[c, h, pl.ds(s * RPT, RPT)])


def _out_body(s_ref, inv_ref, ws_ref, wd_ref, bs_ref, bd_ref, o_ref):
    s0 = lax.concatenate([s_ref[0, 0], s_ref[0, 1]], 1)
    s1 = lax.concatenate([s_ref[1, 0], s_ref[1, 1]], 1)
    inv = inv_ref[...]
    y0 = lax.dot_general(s0, ws_ref[...], (((1,), (1,)), ((), ())),
                         preferred_element_type=jnp.float32)
    y1 = lax.dot_general(s1, wd_ref[...], (((1,), (1,)), ((), ())),
                         preferred_element_type=jnp.float32)
    o_ref[...] = (ALPHA * inv[0][:, None] * y0
                  + (1.0 - ALPHA) * inv[1][:, None] * y1
                  + ALPHA * bs_ref[...] + (1.0 - ALPHA) * bd_ref[...])


_outk = pl.pallas_call(
    _out_body,
    grid=(NPAD // _BLK,),
    in_specs=[
        pl.BlockSpec((2, 2, _BLK, DH), lambda i: (0, 0, i, 0)),
        pl.BlockSpec((2, _BLK), lambda i: (0, i)),
        pl.BlockSpec((D, D), lambda i: (0, 0)),
        pl.BlockSpec((D, D), lambda i: (0, 0)),
        pl.BlockSpec((1, D), lambda i: (0, 0)),
        pl.BlockSpec((1, D), lambda i: (0, 0)),
    ],
    out_specs=pl.BlockSpec((_BLK, D), lambda i: (i, 0)),
    out_shape=jax.ShapeDtypeStruct((NPAD, D), jnp.float32),
)


def kernel(x, edge_index, W_src, b_src, W_dst, b_dst):
    x_pad = jnp.pad(x, ((0, NPAD - N), (0, 0)))
    npad_e = EPAD - E
    sidx_flat = jnp.pad(edge_index, ((0, 0), (0, npad_e)),
                        constant_values=NPAD - 1)
    gsrc = jnp.pad(edge_index[::-1], ((0, 0), (0, npad_e)))
    gidx_flat = 2 * (gsrc + jnp.array([[0], [NPAD]], jnp.int32))
    sidx3 = sidx_flat.reshape(2, NCHS, MCS)
    gidx3 = gidx_flat.reshape(2, NCHS, MCS)
    dpart = _hist(sidx_flat.reshape(2, NCHM, MC))
    inv, t = _scale(dpart, x_pad)
    tab = t.reshape(4 * NPAD, DH)
    svals = _scat(tab, gidx3, sidx3)
    out = _outk(svals, inv, W_src, W_dst,
                b_src.reshape(1, D), b_dst.reshape(1, D))
    return out[:N]
